# Initial kernel scaffold; baseline (speedup 1.0000x reference)
import jax
import jax.numpy as jnp
from jax.experimental import pallas as pl


def kernel(mention_hidden_state, entity_hidden_state, sent_hidden_state, virtual_hidden_state, graph, node_type_table, W):
    raise NotImplementedError("write your pallas kernel here")


# speedup vs baseline: 4.6477x; 4.6477x over previous
import functools
import math

import jax
import jax.numpy as jnp
from jax import lax
from jax.experimental import pallas as pl
from jax.experimental.pallas import tpu as pltpu
from jax.experimental.pallas import tpu_sc as plsc

B = 2
NM, NE_, NS_, NV_ = 2400, 1600, 800, 200
D, DE = 96, 32
H = D + DE
NUM_NODE = NM + NE_ + NS_ + NV_
N = B * NUM_NODE
E = 320000
ALPHA = 0.1
LAMBDA = 0.5
L = 4

NC = 2
NSC = 16
NW = NC * NSC

CH = 128
CPT = 80
EPT = CPT * CH
EPAD = NW * EPT
NP = 10240
RPT = NP // NSC
TRASH = N

BR = 400

def _deg_body(edges, zeros, degout, ib0, ib1, ib2, ib3, ones_rows, agg,
              isem0, isem1, isem2, isem3):
    c = lax.axis_index("c")
    s = lax.axis_index("s")
    ibs = (ib0, ib1, ib2, ib3)
    isems = (isem0, isem1, isem2, isem3)
    ones16 = jnp.ones((16,), jnp.float32)

    @pl.loop(0, CH)
    def _ones(i):
        for k in range(H // 16):
            ones_rows[i, pl.ds(k * 16, 16)] = ones16

    pltpu.sync_copy(zeros.at[pl.ds(s * RPT, RPT)], agg.at[pl.ds(s * RPT, RPT)])
    for t in range(4):
        pltpu.async_copy(edges.at[c, s, t], ibs[t], isems[t])
    plsc.subcore_barrier()

    @pl.loop(0, CPT - 4, step=4)
    def _chunk(j):
        for t in range(4):
            m = j + t
            ib, isem = ibs[t], isems[t]
            pltpu.make_async_copy(edges.at[c, s, m], ib, isem).wait()
            pltpu.sync_copy(ones_rows, agg.at[ib.at[1]], add=True)
            pltpu.async_copy(edges.at[c, s, m + 4], ib, isem)

    for t in range(4):
        m = CPT - 4 + t
        ib, isem = ibs[t], isems[t]
        pltpu.make_async_copy(edges.at[c, s, m], ib, isem).wait()
        pltpu.sync_copy(ones_rows, agg.at[ib.at[1]], add=True)

    plsc.subcore_barrier()
    pltpu.sync_copy(agg.at[pl.ds(s * RPT, RPT)], degout.at[c, pl.ds(s * RPT, RPT)])


def _scatter_body(hs, edges, zeros, out,
                  ib0, ib1, ib2, ib3, rows0, rows1, agg,
                  isem0, isem1, isem2, isem3, gsem0, gsem1):
    c = lax.axis_index("c")
    s = lax.axis_index("s")
    ibs = (ib0, ib1, ib2, ib3)
    isems = (isem0, isem1, isem2, isem3)
    rbs = (rows0, rows1)
    gsems = (gsem0, gsem1)

    pltpu.sync_copy(zeros.at[pl.ds(s * RPT, RPT)], agg.at[pl.ds(s * RPT, RPT)])

    for t in range(4):
        pltpu.async_copy(edges.at[c, s, t], ibs[t], isems[t])
    for t in range(2):
        pltpu.make_async_copy(edges.at[c, s, t], ibs[t], isems[t]).wait()
        pltpu.async_copy(hs.at[ibs[t].at[0]], rbs[t], gsems[t])

    plsc.subcore_barrier()

    @pl.loop(0, CPT - 4, step=4)
    def _chunk(j):
        for t in range(4):
            m = j + t
            ib, isem = ibs[t], isems[t]
            rb, gsem = rbs[t % 2], gsems[t % 2]
            ib2_, isem2_ = ibs[(t + 2) % 4], isems[(t + 2) % 4]
            pltpu.make_async_copy(hs.at[ib.at[0]], rb, gsem).wait()
            pltpu.sync_copy(rb, agg.at[ib.at[1]], add=True)
            pltpu.async_copy(edges.at[c, s, m + 4], ib, isem)
            pltpu.make_async_copy(edges.at[c, s, m + 2], ib2_, isem2_).wait()
            pltpu.async_copy(hs.at[ib2_.at[0]], rb, gsem)

    for t in range(4):
        m = CPT - 4 + t
        ib = ibs[t]
        rb, gsem = rbs[t % 2], gsems[t % 2]
        pltpu.make_async_copy(hs.at[ib.at[0]], rb, gsem).wait()
        pltpu.sync_copy(rb, agg.at[ib.at[1]], add=True)
        if t < 2:
            ib2_, isem2_ = ibs[t + 2], isems[t + 2]
            pltpu.make_async_copy(edges.at[c, s, m + 2], ib2_, isem2_).wait()
            pltpu.async_copy(hs.at[ib2_.at[0]], rb, gsem)

    plsc.subcore_barrier()
    pltpu.sync_copy(agg.at[pl.ds(s * RPT, RPT)], out.at[c, pl.ds(s * RPT, RPT)])


@functools.cache
def _sc_calls():
    mesh = plsc.VectorSubcoreMesh(
        core_axis_name="c", subcore_axis_name="s", num_cores=NC, num_subcores=NSC
    )
    deg = pl.kernel(
        _deg_body,
        out_type=jax.ShapeDtypeStruct((NC, NP, H), jnp.float32),
        mesh=mesh,
        scratch_types=[
            pltpu.VMEM((2, CH), jnp.int32),
            pltpu.VMEM((2, CH), jnp.int32),
            pltpu.VMEM((2, CH), jnp.int32),
            pltpu.VMEM((2, CH), jnp.int32),
            pltpu.VMEM((CH, H), jnp.float32),
            pltpu.VMEM_SHARED((NP, H), jnp.float32),
            pltpu.SemaphoreType.DMA,
            pltpu.SemaphoreType.DMA,
            pltpu.SemaphoreType.DMA,
            pltpu.SemaphoreType.DMA,
        ],
    )
    scatter = pl.kernel(
        _scatter_body,
        out_type=jax.ShapeDtypeStruct((NC, NP, H), jnp.float32),
        mesh=mesh,
        scratch_types=[
            pltpu.VMEM((2, CH), jnp.int32),
            pltpu.VMEM((2, CH), jnp.int32),
            pltpu.VMEM((2, CH), jnp.int32),
            pltpu.VMEM((2, CH), jnp.int32),
            pltpu.VMEM((CH, H), jnp.float32),
            pltpu.VMEM((CH, H), jnp.float32),
            pltpu.VMEM_SHARED((NP, H), jnp.float32),
            pltpu.SemaphoreType.DMA,
            pltpu.SemaphoreType.DMA,
            pltpu.SemaphoreType.DMA,
            pltpu.SemaphoreType.DMA,
            pltpu.SemaphoreType.DMA,
            pltpu.SemaphoreType.DMA,
        ],
    )
    return deg, scatter


def _prep_body(deg_ref, x_ref, g_ref, dinv_ref, hs_ref):
    d = deg_ref[0, :, 0:1] + deg_ref[1, :, 0:1] + 1.0
    g = lax.rsqrt(d)
    gb = jnp.broadcast_to(g, (BR, H))
    g_ref[...] = gb
    dinv_ref[...] = jnp.broadcast_to(1.0 / d, (BR, H))
    hs_ref[...] = x_ref[...] * gb


_prep_call = pl.pallas_call(
    _prep_body,
    grid=(N // BR,),
    in_specs=[
        pl.BlockSpec((NC, BR, H), lambda i: (0, i, 0)),
        pl.BlockSpec((BR, H), lambda i: (i, 0)),
    ],
    out_specs=[pl.BlockSpec((BR, H), lambda i: (i, 0))] * 3,
    out_shape=[jax.ShapeDtypeStruct((N, H), jnp.float32)] * 3,
)


def _layer_body(beta, s_ref, h_ref, h0_ref, g_ref, dinv_ref, w_ref, hn_ref, hsn_ref):
    s2 = s_ref[0] + s_ref[1]
    g = g_ref[...]
    agg = g * s2 + h_ref[...] * dinv_ref[...]
    sup = (1.0 - ALPHA) * agg + ALPHA * h0_ref[...]
    z = (1.0 - beta) * sup + beta * jnp.dot(
        sup, w_ref[...], preferred_element_type=jnp.float32
    )
    hn = jnp.maximum(z, 0.0)
    hn_ref[...] = hn
    hsn_ref[...] = hn * g


def _make_layer(beta):
    return pl.pallas_call(
        functools.partial(_layer_body, beta),
        grid=(N // BR,),
        in_specs=[
            pl.BlockSpec((NC, BR, H), lambda i: (0, i, 0)),
            pl.BlockSpec((BR, H), lambda i: (i, 0)),
            pl.BlockSpec((BR, H), lambda i: (i, 0)),
            pl.BlockSpec((BR, H), lambda i: (i, 0)),
            pl.BlockSpec((BR, H), lambda i: (i, 0)),
            pl.BlockSpec((H, H), lambda i: (0, 0)),
        ],
        out_specs=[pl.BlockSpec((BR, H), lambda i: (i, 0))] * 2,
        out_shape=[jax.ShapeDtypeStruct((N, H), jnp.float32)] * 2,
    )


_layer_calls = [_make_layer(float(math.log(LAMBDA / (l + 1) + 1.0))) for l in range(L)]


def kernel(mention_hidden_state, entity_hidden_state, sent_hidden_state,
           virtual_hidden_state, graph, node_type_table, W):
    def add_type(hs, type_id, n):
        emb = jnp.broadcast_to(node_type_table[type_id][None, None, :], (B, n, DE))
        return jnp.concatenate([hs, emb], axis=2)

    node = jnp.concatenate(
        [
            add_type(mention_hidden_state, 0, NM),
            add_type(entity_hidden_state, 1, NE_),
            add_type(sent_hidden_state, 2, NS_),
            add_type(virtual_hidden_state, 3, NV_),
        ],
        axis=1,
    ).reshape(N, H)

    src_p = jnp.concatenate(
        [graph[0], jnp.zeros((EPAD - E,), jnp.int32)]
    ).reshape(NC, NSC, CPT, CH)
    dst_p = jnp.concatenate(
        [graph[1], jnp.full((EPAD - E,), TRASH, jnp.int32)]
    ).reshape(NC, NSC, CPT, CH)
    edges = jnp.stack([src_p, dst_p], axis=3)
    zeros = jnp.zeros((NP, H), jnp.float32)

    _deg_call, _scatter_call = _sc_calls()
    degp = _deg_call(edges, zeros)
    g, dinv, hs = _prep_call(degp, node)

    h = node
    h0 = node
    for l in range(L):
        S = _scatter_call(hs, edges, zeros)
        h, hs = _layer_calls[l](S, h, h0, g, dinv, W[l])

    out3 = h.reshape(B, NUM_NODE, H)
    return out3[:, NM:NM + NE_], h


# speedup vs baseline: 5.0413x; 1.0847x over previous
import functools
import math

import jax
import jax.numpy as jnp
from jax import lax
from jax.experimental import pallas as pl
from jax.experimental.pallas import tpu as pltpu
from jax.experimental.pallas import tpu_sc as plsc

B = 2
NM, NE_, NS_, NV_ = 2400, 1600, 800, 200
D, DE = 96, 32
H = D + DE
NUM_NODE = NM + NE_ + NS_ + NV_
N = B * NUM_NODE
E = 320000
ALPHA = 0.1
LAMBDA = 0.5
L = 4

NC = 2
NSC = 16
NW = NC * NSC

CH = 64
CPT = 160
NRB = 4
NIB = 8
EPT = CPT * CH
EPAD = NW * EPT
NP = 10240
RPT = NP // NSC
TRASH = N

BR = 400

def _deg_body(edges, zeros, degout, ib0, ib1, ib2, ib3, ones_rows, agg,
              isem0, isem1, isem2, isem3):
    c = lax.axis_index("c")
    s = lax.axis_index("s")
    ibs = (ib0, ib1, ib2, ib3)
    isems = (isem0, isem1, isem2, isem3)
    ones16 = jnp.ones((16,), jnp.float32)

    @pl.loop(0, CH)
    def _ones(i):
        for k in range(H // 16):
            ones_rows[i, pl.ds(k * 16, 16)] = ones16

    pltpu.sync_copy(zeros.at[pl.ds(s * RPT, RPT)], agg.at[pl.ds(s * RPT, RPT)])
    for t in range(4):
        pltpu.async_copy(edges.at[c, s, t], ibs[t], isems[t])
    plsc.subcore_barrier()

    @pl.loop(0, CPT - 4, step=4)
    def _chunk(j):
        for t in range(4):
            m = j + t
            ib, isem = ibs[t], isems[t]
            pltpu.make_async_copy(edges.at[c, s, m], ib, isem).wait()
            pltpu.sync_copy(ones_rows, agg.at[ib.at[1]], add=True)
            pltpu.async_copy(edges.at[c, s, m + 4], ib, isem)

    for t in range(4):
        m = CPT - 4 + t
        ib, isem = ibs[t], isems[t]
        pltpu.make_async_copy(edges.at[c, s, m], ib, isem).wait()
        pltpu.sync_copy(ones_rows, agg.at[ib.at[1]], add=True)

    plsc.subcore_barrier()
    pltpu.sync_copy(agg.at[pl.ds(s * RPT, RPT)], degout.at[c, pl.ds(s * RPT, RPT)])


def _scatter_body(hs, edges, zeros, out,
                  ib0, ib1, ib2, ib3, ib4, ib5, ib6, ib7,
                  rows0, rows1, rows2, rows3, agg,
                  isem0, isem1, isem2, isem3, isem4, isem5, isem6, isem7,
                  gsem0, gsem1, gsem2, gsem3):
    c = lax.axis_index("c")
    s = lax.axis_index("s")
    ibs = (ib0, ib1, ib2, ib3, ib4, ib5, ib6, ib7)
    isems = (isem0, isem1, isem2, isem3, isem4, isem5, isem6, isem7)
    rbs = (rows0, rows1, rows2, rows3)
    gsems = (gsem0, gsem1, gsem2, gsem3)

    pltpu.sync_copy(zeros.at[pl.ds(s * RPT, RPT)], agg.at[pl.ds(s * RPT, RPT)])

    for t in range(NIB):
        pltpu.async_copy(edges.at[c, s, t], ibs[t], isems[t])
    for t in range(NRB):
        pltpu.make_async_copy(edges.at[c, s, t], ibs[t], isems[t]).wait()
        pltpu.async_copy(hs.at[ibs[t].at[0]], rbs[t], gsems[t])

    plsc.subcore_barrier()

    @pl.loop(0, CPT - NIB, step=NIB)
    def _chunk(j):
        for t in range(NIB):
            m = j + t
            ib, isem = ibs[t], isems[t]
            rb, gsem = rbs[t % NRB], gsems[t % NRB]
            ibn, isemn = ibs[(t + NRB) % NIB], isems[(t + NRB) % NIB]
            pltpu.make_async_copy(hs.at[ib.at[0]], rb, gsem).wait()
            pltpu.sync_copy(rb, agg.at[ib.at[1]], add=True)
            pltpu.async_copy(edges.at[c, s, m + NIB], ib, isem)
            pltpu.make_async_copy(edges.at[c, s, m + NRB], ibn, isemn).wait()
            pltpu.async_copy(hs.at[ibn.at[0]], rb, gsem)

    for t in range(NIB):
        m = CPT - NIB + t
        ib = ibs[t]
        rb, gsem = rbs[t % NRB], gsems[t % NRB]
        pltpu.make_async_copy(hs.at[ib.at[0]], rb, gsem).wait()
        pltpu.sync_copy(rb, agg.at[ib.at[1]], add=True)
        if t < NIB - NRB:
            ibn, isemn = ibs[t + NRB], isems[t + NRB]
            pltpu.make_async_copy(edges.at[c, s, m + NRB], ibn, isemn).wait()
            pltpu.async_copy(hs.at[ibn.at[0]], rb, gsem)

    plsc.subcore_barrier()
    pltpu.sync_copy(agg.at[pl.ds(s * RPT, RPT)], out.at[c, pl.ds(s * RPT, RPT)])


@functools.cache
def _sc_calls():
    mesh = plsc.VectorSubcoreMesh(
        core_axis_name="c", subcore_axis_name="s", num_cores=NC, num_subcores=NSC
    )
    deg = pl.kernel(
        _deg_body,
        out_type=jax.ShapeDtypeStruct((NC, NP, H), jnp.float32),
        mesh=mesh,
        scratch_types=[
            pltpu.VMEM((2, CH), jnp.int32),
            pltpu.VMEM((2, CH), jnp.int32),
            pltpu.VMEM((2, CH), jnp.int32),
            pltpu.VMEM((2, CH), jnp.int32),
            pltpu.VMEM((CH, H), jnp.float32),
            pltpu.VMEM_SHARED((NP, H), jnp.float32),
            pltpu.SemaphoreType.DMA,
            pltpu.SemaphoreType.DMA,
            pltpu.SemaphoreType.DMA,
            pltpu.SemaphoreType.DMA,
        ],
    )
    scatter = pl.kernel(
        _scatter_body,
        out_type=jax.ShapeDtypeStruct((NC, NP, H), jnp.float32),
        mesh=mesh,
        scratch_types=(
            [pltpu.VMEM((2, CH), jnp.int32)] * NIB
            + [pltpu.VMEM((CH, H), jnp.float32)] * NRB
            + [pltpu.VMEM_SHARED((NP, H), jnp.float32)]
            + [pltpu.SemaphoreType.DMA] * (NIB + NRB)
        ),
    )
    return deg, scatter


def _prep_body(deg_ref, x_ref, g_ref, dinv_ref, hs_ref):
    d = deg_ref[0, :, 0:1] + deg_ref[1, :, 0:1] + 1.0
    g = lax.rsqrt(d)
    gb = jnp.broadcast_to(g, (BR, H))
    g_ref[...] = gb
    dinv_ref[...] = jnp.broadcast_to(1.0 / d, (BR, H))
    hs_ref[...] = x_ref[...] * gb


_prep_call = pl.pallas_call(
    _prep_body,
    grid=(N // BR,),
    in_specs=[
        pl.BlockSpec((NC, BR, H), lambda i: (0, i, 0)),
        pl.BlockSpec((BR, H), lambda i: (i, 0)),
    ],
    out_specs=[pl.BlockSpec((BR, H), lambda i: (i, 0))] * 3,
    out_shape=[jax.ShapeDtypeStruct((N, H), jnp.float32)] * 3,
)


def _layer_body(beta, s_ref, h_ref, h0_ref, g_ref, dinv_ref, w_ref, hn_ref, hsn_ref):
    s2 = s_ref[0] + s_ref[1]
    g = g_ref[...]
    agg = g * s2 + h_ref[...] * dinv_ref[...]
    sup = (1.0 - ALPHA) * agg + ALPHA * h0_ref[...]
    z = (1.0 - beta) * sup + beta * jnp.dot(
        sup, w_ref[...], preferred_element_type=jnp.float32
    )
    hn = jnp.maximum(z, 0.0)
    hn_ref[...] = hn
    hsn_ref[...] = hn * g


def _make_layer(beta):
    return pl.pallas_call(
        functools.partial(_layer_body, beta),
        grid=(N // BR,),
        in_specs=[
            pl.BlockSpec((NC, BR, H), lambda i: (0, i, 0)),
            pl.BlockSpec((BR, H), lambda i: (i, 0)),
            pl.BlockSpec((BR, H), lambda i: (i, 0)),
            pl.BlockSpec((BR, H), lambda i: (i, 0)),
            pl.BlockSpec((BR, H), lambda i: (i, 0)),
            pl.BlockSpec((H, H), lambda i: (0, 0)),
        ],
        out_specs=[pl.BlockSpec((BR, H), lambda i: (i, 0))] * 2,
        out_shape=[jax.ShapeDtypeStruct((N, H), jnp.float32)] * 2,
    )


_layer_calls = [_make_layer(float(math.log(LAMBDA / (l + 1) + 1.0))) for l in range(L)]


def kernel(mention_hidden_state, entity_hidden_state, sent_hidden_state,
           virtual_hidden_state, graph, node_type_table, W):
    def add_type(hs, type_id, n):
        emb = jnp.broadcast_to(node_type_table[type_id][None, None, :], (B, n, DE))
        return jnp.concatenate([hs, emb], axis=2)

    node = jnp.concatenate(
        [
            add_type(mention_hidden_state, 0, NM),
            add_type(entity_hidden_state, 1, NE_),
            add_type(sent_hidden_state, 2, NS_),
            add_type(virtual_hidden_state, 3, NV_),
        ],
        axis=1,
    ).reshape(N, H)

    src_p = jnp.concatenate(
        [graph[0], jnp.zeros((EPAD - E,), jnp.int32)]
    ).reshape(NC, NSC, CPT, CH)
    dst_p = jnp.concatenate(
        [graph[1], jnp.full((EPAD - E,), TRASH, jnp.int32)]
    ).reshape(NC, NSC, CPT, CH)
    edges = jnp.stack([src_p, dst_p], axis=3)
    zeros = jnp.zeros((NP, H), jnp.float32)

    _deg_call, _scatter_call = _sc_calls()
    degp = _deg_call(edges, zeros)
    g, dinv, hs = _prep_call(degp, node)

    h = node
    h0 = node
    for l in range(L):
        S = _scatter_call(hs, edges, zeros)
        h, hs = _layer_calls[l](S, h, h0, g, dinv, W[l])

    out3 = h.reshape(B, NUM_NODE, H)
    return out3[:, NM:NM + NE_], h
